# baseline (speedup 1.0000x reference)
import functools

import jax
import jax.numpy as jnp
from jax import lax
from jax.experimental import pallas as pl
from jax.experimental.pallas import tpu as pltpu
from jax.experimental.pallas import tpu_sc as plsc

N = 10000
NP = 10240
D = 128
HD = 64
E = 320000

NC = 2
NS = 16

EP = 327680
B = 128
NB = EP // NS // B
CH = 16
NCH = NB // CH
ROWS_PER_TILE = N // NS
ZR = 125


def _sc_halves(ego0, ego1, src_w, dst_w, val_w):
  mesh = plsc.VectorSubcoreMesh(core_axis_name="c", subcore_axis_name="s")

  @functools.partial(
      pl.kernel,
      out_type=jax.ShapeDtypeStruct((N, D), jnp.float32),
      mesh=mesh,
      scratch_types=[
          pltpu.VMEM((2, CH, B), jnp.int32),
          pltpu.VMEM((2, CH, B), jnp.int32),
          pltpu.VMEM((2, CH, B), jnp.float32),
          pltpu.VMEM((6, B // 2, D), jnp.float32),
          pltpu.VMEM((ZR, HD), jnp.float32),
          pltpu.VMEM_SHARED((N, HD), jnp.float32),
          pltpu.SemaphoreType.DMA((6,)),
          pltpu.SemaphoreType.DMA((6,)),
          pltpu.SemaphoreType.DMA((2,)),
      ],
      compiler_params=pltpu.CompilerParams(use_tc_tiling_on_sc=False),
  )
  def k(ego0_hbm, ego1_hbm, src_hbm, dst_hbm, val_hbm, out_hbm,
        src_v, dst_v, val_v, rows_v, zbuf_v, accum, gsem, ssem, isem):
    c = lax.axis_index("c")
    s = lax.axis_index("s")

    def zrow(i, carry):
      for kk in range(HD // 16):
        zbuf_v[i, pl.ds(kk * 16, 16)] = jnp.zeros((16,), jnp.float32)
      return carry
    lax.fori_loop(0, ZR, zrow, 0)
    for t in range(ROWS_PER_TILE // ZR):
      pltpu.sync_copy(zbuf_v,
                      accum.at[pl.ds(s * ROWS_PER_TILE + t * ZR, ZR)])
    plsc.subcore_barrier()

    def stage_sync(ci, slot):
      pltpu.sync_copy(src_hbm.at[s, pl.ds(ci * CH, CH)], src_v.at[slot])
      pltpu.sync_copy(dst_hbm.at[s, pl.ds(ci * CH, CH)], dst_v.at[slot])
      pltpu.sync_copy(val_hbm.at[s, pl.ds(ci * CH, CH)], val_v.at[slot])

    def stage_async(ci, slot):
      pltpu.async_copy(src_hbm.at[s, pl.ds(ci * CH, CH)], src_v.at[slot],
                       isem.at[slot])
      pltpu.async_copy(dst_hbm.at[s, pl.ds(ci * CH, CH)], dst_v.at[slot],
                       isem.at[slot])
      pltpu.async_copy(val_hbm.at[s, pl.ds(ci * CH, CH)], val_v.at[slot],
                       isem.at[slot])

    def wait_stage(slot):
      for hbm, ref in ((src_hbm, src_v), (dst_hbm, dst_v), (val_hbm, val_v)):
        pltpu.make_async_copy(hbm.at[s, pl.ds(0, CH)], ref.at[slot],
                              isem.at[slot]).wait()

    def issue_gather(t):
      slot = (t // CH) % 2
      buf = t % 6
      idx = src_v.at[slot, t % CH, pl.ds(0, B // 2)]

      @pl.when(c == 0)
      def _():
        pltpu.async_copy(ego0_hbm.at[idx], rows_v.at[buf], gsem.at[buf])

      @pl.when(c == 1)
      def _():
        pltpu.async_copy(ego1_hbm.at[idx], rows_v.at[buf], gsem.at[buf])

    def wait_scatter(buf):
      pltpu.make_async_copy(
          rows_v.at[buf], accum.at[dst_v.at[0, 0]], ssem.at[buf]).wait()

    stage_sync(0, 0)
    issue_gather(0)
    issue_gather(1)
    issue_gather(2)

    def batch(j, carry):
      b = j % 6
      slot = (j // CH) % 2
      bb = j % CH
      nxt = j + 3

      @pl.when(nxt < NB)
      def _():
        @pl.when(nxt % CH == 0)
        def _():
          wait_stage((nxt // CH) % 2)

        issue_gather(nxt)

      @pl.when(jnp.logical_and(j % CH == 3, j // CH + 1 < NCH))
      def _():
        stage_async(j // CH + 1, (j // CH + 1) % 2)

      pltpu.make_async_copy(
          ego0_hbm.at[src_v.at[slot, bb, pl.ds(0, B // 2)]], rows_v.at[b],
          gsem.at[b]).wait()


      return carry
    lax.fori_loop(0, NB, batch, 0)


    plsc.subcore_barrier()

    base = s * ROWS_PER_TILE
    pltpu.sync_copy(accum.at[pl.ds(base, ROWS_PER_TILE)],
                    out_hbm.at[pl.ds(base, ROWS_PER_TILE),
                               pl.ds(c * HD, HD)])

  return k(ego0, ego1, src_w, dst_w, val_w)


def kernel(ego_embeddings, edge_index, edge_vals):
  pad = EP - E
  src_w = jnp.concatenate(
      [edge_index[0], jnp.zeros((pad,), jnp.int32)]).reshape(NS, NB, B)
  dst_w = jnp.concatenate(
      [edge_index[1], jnp.zeros((pad,), jnp.int32)]).reshape(NS, NB, B)
  val_w = jnp.concatenate(
      [edge_vals, jnp.zeros((pad,), jnp.float32)]).reshape(NS, NB, B)
  ego0 = ego_embeddings
  ego1 = ego_embeddings
  return _sc_halves(ego0, ego1, src_w, dst_w, val_w)


# speedup vs baseline: 1.7166x; 1.7166x over previous
import functools

import jax
import jax.numpy as jnp
from jax import lax
from jax.experimental import pallas as pl
from jax.experimental.pallas import tpu as pltpu
from jax.experimental.pallas import tpu_sc as plsc

N = 10000
NP = 10240
D = 128
HD = 64
E = 320000

NC = 2
NS = 16

EP = 327680
B = 128
NB = EP // NS // B
CH = 16
NCH = NB // CH
ROWS_PER_TILE = N // NS
ZR = 125


def _sc_halves(ego0, ego1, src_w, dst_w, val_w):
  mesh = plsc.VectorSubcoreMesh(core_axis_name="c", subcore_axis_name="s")

  @functools.partial(
      pl.kernel,
      out_type=jax.ShapeDtypeStruct((N, D), jnp.float32),
      mesh=mesh,
      scratch_types=[
          pltpu.VMEM((2, CH, B), jnp.int32),
          pltpu.VMEM((2, CH, B), jnp.int32),
          pltpu.VMEM((2, CH, B), jnp.float32),
          pltpu.VMEM((6, B, HD // 2), jnp.int32),
          pltpu.VMEM((4, B, HD), jnp.float32),
          pltpu.VMEM_SHARED((N, HD), jnp.float32),
          pltpu.SemaphoreType.DMA((6,)),
          pltpu.SemaphoreType.DMA((4,)),
          pltpu.SemaphoreType.DMA((2,)),
      ],
      compiler_params=pltpu.CompilerParams(use_tc_tiling_on_sc=False,
                                           needs_layout_passes=False),
  )
  def k(ego0_hbm, ego1_hbm, src_hbm, dst_hbm, val_hbm, out_hbm,
        src_v, dst_v, val_v, rows_v, srow_v, accum, gsem, ssem, isem):
    c = lax.axis_index("c")
    s = lax.axis_index("s")

    def zrow(i, carry):
      for kk in range(HD // 16):
        srow_v[0, i, pl.ds(kk * 16, 16)] = jnp.zeros((16,), jnp.float32)
      return carry
    lax.fori_loop(0, ZR, zrow, 0)
    for t in range(ROWS_PER_TILE // ZR):
      pltpu.sync_copy(srow_v.at[0, pl.ds(0, ZR)],
                      accum.at[pl.ds(s * ROWS_PER_TILE + t * ZR, ZR)])
    plsc.subcore_barrier()

    def stage_sync(ci, slot):
      pltpu.sync_copy(src_hbm.at[s, pl.ds(ci * CH, CH)], src_v.at[slot])
      pltpu.sync_copy(dst_hbm.at[s, pl.ds(ci * CH, CH)], dst_v.at[slot])
      pltpu.sync_copy(val_hbm.at[s, pl.ds(ci * CH, CH)], val_v.at[slot])

    def stage_async(ci, slot):
      pltpu.async_copy(src_hbm.at[s, pl.ds(ci * CH, CH)], src_v.at[slot],
                       isem.at[slot])
      pltpu.async_copy(dst_hbm.at[s, pl.ds(ci * CH, CH)], dst_v.at[slot],
                       isem.at[slot])
      pltpu.async_copy(val_hbm.at[s, pl.ds(ci * CH, CH)], val_v.at[slot],
                       isem.at[slot])

    def wait_stage(slot):
      for hbm, ref in ((src_hbm, src_v), (dst_hbm, dst_v), (val_hbm, val_v)):
        pltpu.make_async_copy(hbm.at[s, pl.ds(0, CH)], ref.at[slot],
                              isem.at[slot]).wait()

    def issue_gather(t):
      slot = (t // CH) % 2
      buf = t % 6
      idx = src_v.at[slot, t % CH]

      @pl.when(c == 0)
      def _():
        pltpu.async_copy(ego0_hbm.at[idx], rows_v.at[buf], gsem.at[buf])

      @pl.when(c == 1)
      def _():
        pltpu.async_copy(ego1_hbm.at[idx], rows_v.at[buf], gsem.at[buf])

    def wait_scatter(buf):
      pltpu.make_async_copy(
          srow_v.at[buf], accum.at[dst_v.at[0, 0]], ssem.at[buf]).wait()

    stage_sync(0, 0)
    issue_gather(0)
    issue_gather(1)
    issue_gather(2)

    def batch(j, carry):
      b = j % 6
      sb = j % 4
      slot = (j // CH) % 2
      bb = j % CH
      nxt = j + 3

      @pl.when(nxt < NB)
      def _():
        @pl.when(nxt % CH == 0)
        def _():
          wait_stage((nxt // CH) % 2)
        issue_gather(nxt)

      @pl.when(jnp.logical_and(j % CH == 3, j // CH + 1 < NCH))
      def _():
        stage_async(j // CH + 1, (j // CH + 1) % 2)

      pltpu.make_async_copy(
          ego0_hbm.at[src_v.at[slot, bb]], rows_v.at[b],
          gsem.at[b]).wait()

      @pl.when(j >= 4)
      def _():
        wait_scatter(sb)

      shamt = jnp.full((16,), 16, jnp.int32)
      mask = jnp.full((16,), -65536, jnp.int32)

      @plsc.parallel_loop(0, B // 16, unroll=2)
      def group(g):
        vals16 = val_v[slot, bb, pl.ds(g * 16, 16)]
        base = g * 16
        for e in range(16):
          v = vals16[e]
          for blk in range(HD // 32):
            w = rows_v[b, base + e, pl.ds(blk * 16, 16)]
            lo = plsc.bitcast(lax.shift_left(w, shamt), jnp.float32)
            hi = plsc.bitcast(lax.bitwise_and(w, mask), jnp.float32)
            srow_v[sb, base + e, pl.ds(blk * 32, 16)] = lo * v
            srow_v[sb, base + e, pl.ds(blk * 32 + 16, 16)] = hi * v

      pltpu.async_copy(srow_v.at[sb], accum.at[dst_v.at[slot, bb]],
                       ssem.at[sb], add=True)
      return carry
    lax.fori_loop(0, NB, batch, 0)

    for buf in range(4):
      wait_scatter(buf)

    plsc.subcore_barrier()

    base = s * ROWS_PER_TILE
    pltpu.sync_copy(accum.at[pl.ds(base, ROWS_PER_TILE)],
                    out_hbm.at[pl.ds(base, ROWS_PER_TILE),
                               pl.ds(c * HD, HD)])

  return k(ego0, ego1, src_w, dst_w, val_w)


def kernel(ego_embeddings, edge_index, edge_vals):
  pad = EP - E
  src_w = jnp.concatenate(
      [edge_index[0], jnp.zeros((pad,), jnp.int32)]).reshape(NS, NB, B)
  dst_w = jnp.concatenate(
      [edge_index[1], jnp.zeros((pad,), jnp.int32)]).reshape(NS, NB, B)
  val_w = jnp.concatenate(
      [edge_vals, jnp.zeros((pad,), jnp.float32)]).reshape(NS, NB, B)
  def pack_half(h):
    hp = h.reshape(N, 2, 2, 16).transpose(0, 1, 3, 2).reshape(N, 32, 2)
    return jax.lax.bitcast_convert_type(
        hp.astype(jnp.bfloat16), jnp.int32)

  ego0 = pack_half(ego_embeddings[:, :HD])
  ego1 = pack_half(ego_embeddings[:, HD:])
  return _sc_halves(ego0, ego1, src_w, dst_w, val_w)


# speedup vs baseline: 1.7899x; 1.0427x over previous
import functools

import jax
import jax.numpy as jnp
from jax import lax
from jax.experimental import pallas as pl
from jax.experimental.pallas import tpu as pltpu
from jax.experimental.pallas import tpu_sc as plsc

N = 10000
NP = 10240
D = 128
HD = 64
E = 320000

NC = 2
NS = 16

EP = 327680
B = 128
NB = EP // NS // B
CH = 16
NCH = NB // CH
ROWS_PER_TILE = N // NS
ZR = 125


def _sc_halves(ego0, ego1, src_w, dst_w, val_w):
  mesh = plsc.VectorSubcoreMesh(core_axis_name="c", subcore_axis_name="s")

  @functools.partial(
      pl.kernel,
      out_type=jax.ShapeDtypeStruct((N, D), jnp.float32),
      mesh=mesh,
      scratch_types=[
          pltpu.VMEM((2, CH, B), jnp.int32),
          pltpu.VMEM((2, CH, B), jnp.int32),
          pltpu.VMEM((2, CH, B), jnp.float32),
          pltpu.VMEM((6, B, HD // 2), jnp.int32),
          pltpu.VMEM((4, B, HD), jnp.float32),
          pltpu.VMEM_SHARED((N, HD), jnp.float32),
          pltpu.SemaphoreType.DMA((6,)),
          pltpu.SemaphoreType.DMA((4,)),
          pltpu.SemaphoreType.DMA((2,)),
      ],
      compiler_params=pltpu.CompilerParams(use_tc_tiling_on_sc=False,
                                           needs_layout_passes=False),
  )
  def k(ego0_hbm, ego1_hbm, src_hbm, dst_hbm, val_hbm, out_hbm,
        src_v, dst_v, val_v, rows_v, srow_v, accum, gsem, ssem, isem):
    c = lax.axis_index("c")
    s = lax.axis_index("s")

    def zrow(i, carry):
      for kk in range(HD // 16):
        srow_v[0, i, pl.ds(kk * 16, 16)] = jnp.zeros((16,), jnp.float32)
      return carry
    lax.fori_loop(0, ZR, zrow, 0)
    zdescs = [
        pltpu.async_copy(srow_v.at[0, pl.ds(0, ZR)],
                         accum.at[pl.ds(s * ROWS_PER_TILE + t * ZR, ZR)],
                         isem.at[0])
        for t in range(ROWS_PER_TILE // ZR)]
    for d in zdescs:
      d.wait()
    plsc.subcore_barrier()

    def stage_sync(ci, slot):
      pltpu.sync_copy(src_hbm.at[s, pl.ds(ci * CH, CH)], src_v.at[slot])
      pltpu.sync_copy(dst_hbm.at[s, pl.ds(ci * CH, CH)], dst_v.at[slot])
      pltpu.sync_copy(val_hbm.at[s, pl.ds(ci * CH, CH)], val_v.at[slot])

    def stage_async(ci, slot):
      pltpu.async_copy(src_hbm.at[s, pl.ds(ci * CH, CH)], src_v.at[slot],
                       isem.at[slot])
      pltpu.async_copy(dst_hbm.at[s, pl.ds(ci * CH, CH)], dst_v.at[slot],
                       isem.at[slot])
      pltpu.async_copy(val_hbm.at[s, pl.ds(ci * CH, CH)], val_v.at[slot],
                       isem.at[slot])

    def wait_stage(slot):
      for hbm, ref in ((src_hbm, src_v), (dst_hbm, dst_v), (val_hbm, val_v)):
        pltpu.make_async_copy(hbm.at[s, pl.ds(0, CH)], ref.at[slot],
                              isem.at[slot]).wait()

    def issue_gather(t):
      slot = (t // CH) % 2
      buf = t % 6
      idx = src_v.at[slot, t % CH]

      @pl.when(c == 0)
      def _():
        pltpu.async_copy(ego0_hbm.at[idx], rows_v.at[buf], gsem.at[buf])

      @pl.when(c == 1)
      def _():
        pltpu.async_copy(ego1_hbm.at[idx], rows_v.at[buf], gsem.at[buf])

    def wait_scatter(buf):
      pltpu.make_async_copy(
          srow_v.at[buf], accum.at[dst_v.at[0, 0]], ssem.at[buf]).wait()

    stage_sync(0, 0)
    issue_gather(0)
    issue_gather(1)
    issue_gather(2)
    issue_gather(3)

    def batch(j, carry):
      b = j % 6
      sb = j % 4
      slot = (j // CH) % 2
      bb = j % CH
      nxt = j + 4

      @pl.when(nxt < NB)
      def _():
        @pl.when(nxt % CH == 0)
        def _():
          wait_stage((nxt // CH) % 2)
        issue_gather(nxt)

      @pl.when(jnp.logical_and(j % CH == 3, j // CH + 1 < NCH))
      def _():
        stage_async(j // CH + 1, (j // CH + 1) % 2)

      pltpu.make_async_copy(
          ego0_hbm.at[src_v.at[slot, bb]], rows_v.at[b],
          gsem.at[b]).wait()

      @pl.when(j >= 4)
      def _():
        wait_scatter(sb)

      shamt = jnp.full((16,), 16, jnp.int32)
      mask = jnp.full((16,), -65536, jnp.int32)

      @plsc.parallel_loop(0, B // 16, unroll=2)
      def group(g):
        vals16 = val_v[slot, bb, pl.ds(g * 16, 16)]
        base = g * 16
        for e in range(16):
          v = vals16[e]
          for blk in range(HD // 32):
            w = rows_v[b, base + e, pl.ds(blk * 16, 16)]
            lo = plsc.bitcast(lax.shift_left(w, shamt), jnp.float32)
            hi = plsc.bitcast(lax.bitwise_and(w, mask), jnp.float32)
            srow_v[sb, base + e, pl.ds(blk * 32, 16)] = lo * v
            srow_v[sb, base + e, pl.ds(blk * 32 + 16, 16)] = hi * v

      pltpu.async_copy(srow_v.at[sb], accum.at[dst_v.at[slot, bb]],
                       ssem.at[sb], add=True)
      return carry
    lax.fori_loop(0, NB, batch, 0)

    for buf in range(4):
      wait_scatter(buf)

    plsc.subcore_barrier()

    base = s * ROWS_PER_TILE
    pltpu.sync_copy(accum.at[pl.ds(base, ROWS_PER_TILE)],
                    out_hbm.at[pl.ds(base, ROWS_PER_TILE),
                               pl.ds(c * HD, HD)])

  return k(ego0, ego1, src_w, dst_w, val_w)


def kernel(ego_embeddings, edge_index, edge_vals):
  pad = EP - E
  ei = jnp.pad(edge_index, ((0, 0), (0, pad)))
  src_w = ei[0].reshape(NS, NB, B)
  dst_w = ei[1].reshape(NS, NB, B)
  val_w = jnp.pad(edge_vals, (0, pad)).reshape(NS, NB, B)
  def pack_half(h):
    hp = h.reshape(N, 2, 2, 16).transpose(0, 1, 3, 2).reshape(N, 32, 2)
    return jax.lax.bitcast_convert_type(
        hp.astype(jnp.bfloat16), jnp.int32)

  ego0 = pack_half(ego_embeddings[:, :HD])
  ego1 = pack_half(ego_embeddings[:, HD:])
  return _sc_halves(ego0, ego1, src_w, dst_w, val_w)


# speedup vs baseline: 1.7910x; 1.0006x over previous
import functools

import jax
import jax.numpy as jnp
from jax import lax
from jax.experimental import pallas as pl
from jax.experimental.pallas import tpu as pltpu
from jax.experimental.pallas import tpu_sc as plsc

N = 10000
NP = 10240
D = 128
HD = 64
E = 320000

NC = 2
NS = 16

EP = 327680
B = 128
NB = EP // NS // B
CH = 16
NCH = NB // CH
ROWS_PER_TILE = N // NS
ZR = 125


def _sc_halves(ego0, ego1, src_w, dst_w, val_w):
  mesh = plsc.VectorSubcoreMesh(core_axis_name="c", subcore_axis_name="s")

  @functools.partial(
      pl.kernel,
      out_type=jax.ShapeDtypeStruct((N, D), jnp.float32),
      mesh=mesh,
      scratch_types=[
          pltpu.VMEM((2, CH, B), jnp.int32),
          pltpu.VMEM((2, CH, B), jnp.int32),
          pltpu.VMEM((2, CH, B), jnp.float32),
          pltpu.VMEM((6, B, HD // 2), jnp.int32),
          pltpu.VMEM((4, B, HD), jnp.float32),
          pltpu.VMEM_SHARED((N, HD), jnp.float32),
          pltpu.SemaphoreType.DMA((6,)),
          pltpu.SemaphoreType.DMA((4,)),
          pltpu.SemaphoreType.DMA((2,)),
      ],
      compiler_params=pltpu.CompilerParams(use_tc_tiling_on_sc=False,
                                           needs_layout_passes=False),
  )
  def k(ego0_hbm, ego1_hbm, src_hbm, dst_hbm, val_hbm, out_hbm,
        src_v, dst_v, val_v, rows_v, srow_v, accum, gsem, ssem, isem):
    c = lax.axis_index("c")
    s = lax.axis_index("s")

    def zrow(i, carry):
      for kk in range(HD // 16):
        srow_v[0, i, pl.ds(kk * 16, 16)] = jnp.zeros((16,), jnp.float32)
      return carry
    lax.fori_loop(0, ZR, zrow, 0)
    zdescs = [
        pltpu.async_copy(srow_v.at[0, pl.ds(0, ZR)],
                         accum.at[pl.ds(s * ROWS_PER_TILE + t * ZR, ZR)],
                         isem.at[0])
        for t in range(ROWS_PER_TILE // ZR)]
    for d in zdescs:
      d.wait()
    plsc.subcore_barrier()

    def stage_sync(ci, slot):
      pltpu.sync_copy(src_hbm.at[s, pl.ds(ci * CH, CH)], src_v.at[slot])
      pltpu.sync_copy(dst_hbm.at[s, pl.ds(ci * CH, CH)], dst_v.at[slot])
      pltpu.sync_copy(val_hbm.at[s, pl.ds(ci * CH, CH)], val_v.at[slot])

    def stage_async(ci, slot):
      pltpu.async_copy(src_hbm.at[s, pl.ds(ci * CH, CH)], src_v.at[slot],
                       isem.at[slot])
      pltpu.async_copy(dst_hbm.at[s, pl.ds(ci * CH, CH)], dst_v.at[slot],
                       isem.at[slot])
      pltpu.async_copy(val_hbm.at[s, pl.ds(ci * CH, CH)], val_v.at[slot],
                       isem.at[slot])

    def wait_stage(slot):
      for hbm, ref in ((src_hbm, src_v), (dst_hbm, dst_v), (val_hbm, val_v)):
        pltpu.make_async_copy(hbm.at[s, pl.ds(0, CH)], ref.at[slot],
                              isem.at[slot]).wait()

    def issue_gather(t):
      slot = (t // CH) % 2
      buf = t % 6
      idx = src_v.at[slot, t % CH]

      @pl.when(c == 0)
      def _():
        pltpu.async_copy(ego0_hbm.at[idx], rows_v.at[buf], gsem.at[buf])

      @pl.when(c == 1)
      def _():
        pltpu.async_copy(ego1_hbm.at[idx], rows_v.at[buf], gsem.at[buf])

    def wait_scatter(buf):
      pltpu.make_async_copy(
          srow_v.at[buf], accum.at[dst_v.at[0, 0]], ssem.at[buf]).wait()

    stage_sync(0, 0)
    issue_gather(0)
    issue_gather(1)
    issue_gather(2)
    issue_gather(3)
    issue_gather(4)

    def batch(j, carry):
      b = j % 6
      sb = j % 4
      slot = (j // CH) % 2
      bb = j % CH
      nxt = j + 5

      @pl.when(nxt < NB)
      def _():
        @pl.when(nxt % CH == 0)
        def _():
          wait_stage((nxt // CH) % 2)
        issue_gather(nxt)

      @pl.when(jnp.logical_and(j % CH == 3, j // CH + 1 < NCH))
      def _():
        stage_async(j // CH + 1, (j // CH + 1) % 2)

      pltpu.make_async_copy(
          ego0_hbm.at[src_v.at[slot, bb]], rows_v.at[b],
          gsem.at[b]).wait()

      @pl.when(j >= 4)
      def _():
        wait_scatter(sb)

      shamt = jnp.full((16,), 16, jnp.int32)
      mask = jnp.full((16,), -65536, jnp.int32)

      @plsc.parallel_loop(0, B // 16, unroll=4)
      def group(g):
        vals16 = val_v[slot, bb, pl.ds(g * 16, 16)]
        base = g * 16
        for e in range(16):
          v = vals16[e]
          for blk in range(HD // 32):
            w = rows_v[b, base + e, pl.ds(blk * 16, 16)]
            lo = plsc.bitcast(lax.shift_left(w, shamt), jnp.float32)
            hi = plsc.bitcast(lax.bitwise_and(w, mask), jnp.float32)
            srow_v[sb, base + e, pl.ds(blk * 32, 16)] = lo * v
            srow_v[sb, base + e, pl.ds(blk * 32 + 16, 16)] = hi * v

      pltpu.async_copy(srow_v.at[sb], accum.at[dst_v.at[slot, bb]],
                       ssem.at[sb], add=True)
      return carry
    lax.fori_loop(0, NB, batch, 0)

    for buf in range(4):
      wait_scatter(buf)

    plsc.subcore_barrier()

    base = s * ROWS_PER_TILE
    pltpu.sync_copy(accum.at[pl.ds(base, ROWS_PER_TILE)],
                    out_hbm.at[pl.ds(base, ROWS_PER_TILE),
                               pl.ds(c * HD, HD)])

  return k(ego0, ego1, src_w, dst_w, val_w)


def kernel(ego_embeddings, edge_index, edge_vals):
  pad = EP - E
  ei = jnp.pad(edge_index, ((0, 0), (0, pad)))
  src_w = ei[0].reshape(NS, NB, B)
  dst_w = ei[1].reshape(NS, NB, B)
  val_w = jnp.pad(edge_vals, (0, pad)).reshape(NS, NB, B)
  def pack_half(h):
    hp = h.reshape(N, 2, 2, 16).transpose(0, 1, 3, 2).reshape(N, 32, 2)
    return jax.lax.bitcast_convert_type(
        hp.astype(jnp.bfloat16), jnp.int32)

  ego0 = pack_half(ego_embeddings[:, :HD])
  ego1 = pack_half(ego_embeddings[:, HD:])
  return _sc_halves(ego0, ego1, src_w, dst_w, val_w)
